# Initial kernel scaffold; baseline (speedup 1.0000x reference)
import jax
import jax.numpy as jnp
from jax import lax
from jax.experimental import pallas as pl
from jax.experimental.pallas import tpu as pltpu
from jax.experimental.pallas import tpu_sc as plsc

N_NODES = 10000
NPAD = 10240
DIM = 128
DE = 16
E_TOTAL = 320000
NSC = 2
NSUB = 16
NW = NSC * NSUB
EPW = E_TOTAL // NW
C = 128
CT = 16
ROWS_PER_SUB = NPAD // NSUB

_HI = lax.Precision.HIGHEST


def _prep_body(x_ref, ts_ref, w1_ref, w2_ref, bt_ref, fr_ref, ph_ref, h_ref):
    t_enc = jnp.cos(ts_ref[...] * fr_ref[...] + ph_ref[...])
    h = jnp.dot(x_ref[...], w1_ref[...], preferred_element_type=jnp.float32,
                precision=_HI)
    h += jnp.dot(t_enc, w2_ref[...], preferred_element_type=jnp.float32,
                 precision=_HI)
    h_ref[...] = h + bt_ref[...]


def _rst_base_body(h_ref, ws_ref, bs_ref, bn_ref, o_ref):
    o_ref[...] = (jnp.dot(h_ref[...], ws_ref[...],
                          preferred_element_type=jnp.float32, precision=_HI)
                  + bs_ref[...] + bn_ref[...])


def _combine_body(acc_ref, sat_ref, deg_ref, we_ref, be_ref, wn_ref, rb_ref,
                  o_ref, carry_ref):
    i = pl.program_id(0)

    @pl.when(i == 0)
    def _():
        carry_ref[...] = jnp.zeros((1, DIM), jnp.float32)

    a = acc_ref[0] + acc_ref[1]
    s = sat_ref[0] + sat_ref[1]
    deg = deg_ref[0, :, 0:1] + deg_ref[1, :, 0:1]
    h_ns = a + jnp.dot(s, we_ref[...], preferred_element_type=jnp.float32,
                       precision=_HI) + deg * be_ref[...]
    b = h_ns.shape[0]
    r = lax.broadcasted_iota(jnp.int32, (b, b), 0)
    c = lax.broadcasted_iota(jnp.int32, (b, b), 1)
    tril = (r >= c).astype(jnp.float32)
    cs = jnp.dot(tril, h_ns, preferred_element_type=jnp.float32,
                 precision=_HI) + carry_ref[...]
    carry_ref[...] = cs[b - 1:b, :]
    h_neigh = cs / jnp.maximum(deg, 1.0)
    o_ref[...] = rb_ref[...] + jnp.dot(h_neigh, wn_ref[...],
                                       preferred_element_type=jnp.float32,
                                       precision=_HI)


def _sc_body(h_hbm, src_hbm, dst_hbm, attr_hbm,
             acc_out, sat_out, deg_out,
             src_v, dst_v, src_t, dst_t, rows_v, rows_t, attr_v, attr_t,
             ones_v, acc_s, sat_s, dega_s, sem):
    core = lax.axis_index("c")
    sub = lax.axis_index("s")
    wid = core * NSUB + sub
    z16 = jnp.zeros((16,), jnp.float32)
    o16 = jnp.ones((16,), jnp.float32)

    @pl.loop(0, C)
    def _(r):
        @pl.loop(0, DIM, step=16)
        def _(j):
            rows_v[r, pl.ds(j, 16)] = z16

    @pl.loop(0, C)
    def _(r):
        attr_v[r, pl.ds(0, 16)] = z16
        ones_v[r, pl.ds(0, 16)] = o16

    rbase = sub * ROWS_PER_SUB

    @pl.loop(0, ROWS_PER_SUB, step=C)
    def _(k):
        pltpu.sync_copy(rows_v, acc_s.at[pl.ds(rbase + k, C)])
        pltpu.sync_copy(attr_v, sat_s.at[pl.ds(rbase + k, C)])
        pltpu.sync_copy(attr_v, dega_s.at[pl.ds(rbase + k, C)])

    plsc.subcore_barrier()

    ebase = wid * EPW

    def chunk(b, n, s_v, d_v, r_v, a_v):
        pltpu.sync_copy(src_hbm.at[pl.ds(b, n)], s_v)
        pltpu.sync_copy(dst_hbm.at[pl.ds(b, n)], d_v)
        pltpu.sync_copy(attr_hbm.at[pl.ds(b, n)], a_v)
        pltpu.async_copy(h_hbm.at[s_v], r_v, sem).wait()
        pltpu.sync_copy(r_v, acc_s.at[d_v], add=True)
        pltpu.sync_copy(a_v, sat_s.at[d_v], add=True)
        pltpu.sync_copy(ones_v.at[pl.ds(0, n)], dega_s.at[d_v], add=True)

    @pl.loop(0, EPW - CT, step=C)
    def _(i):
        chunk(ebase + i, C, src_v, dst_v, rows_v, attr_v)

    chunk(ebase + (EPW - CT), CT, src_t, dst_t, rows_t, attr_t)

    plsc.subcore_barrier()

    pltpu.sync_copy(acc_s.at[pl.ds(rbase, ROWS_PER_SUB)],
                    acc_out.at[core, pl.ds(rbase, ROWS_PER_SUB)])
    pltpu.sync_copy(sat_s.at[pl.ds(rbase, ROWS_PER_SUB)],
                    sat_out.at[core, pl.ds(rbase, ROWS_PER_SUB)])
    pltpu.sync_copy(dega_s.at[pl.ds(rbase, ROWS_PER_SUB)],
                    deg_out.at[core, pl.ds(rbase, ROWS_PER_SUB)])


def _full(u):
    return pl.BlockSpec(u, lambda *_: tuple(0 for _ in u))


def kernel(x, timestamps, edge_index, edge_attr, new_node_ids,
           time_freq, time_phase, W_time, b_time,
           W_edge, b_edge, W_self, b_self, W_neigh, b_neigh):
    del new_node_ids

    f32 = jnp.float32
    npad = NPAD - N_NODES
    x_p = jnp.pad(x, ((0, npad), (0, 0)))
    ts_p = jnp.pad(jnp.broadcast_to(timestamps[:, None], (N_NODES, DIM)),
                   ((0, npad), (0, 0)))
    src = edge_index[0]
    dst = edge_index[1]
    w1 = W_time[:DIM]
    w2 = W_time[DIM:]

    blk = 1024
    grid1 = NPAD // blk
    h_self = pl.pallas_call(
        _prep_body,
        grid=(grid1,),
        in_specs=[
            pl.BlockSpec((blk, DIM), lambda i: (i, 0)),
            pl.BlockSpec((blk, DIM), lambda i: (i, 0)),
            _full((DIM, DIM)), _full((DIM, DIM)),
            _full((1, DIM)), _full((1, DIM)), _full((1, DIM)),
        ],
        out_specs=pl.BlockSpec((blk, DIM), lambda i: (i, 0)),
        out_shape=jax.ShapeDtypeStruct((NPAD, DIM), f32),
    )(x_p, ts_p, w1, w2, b_time[None, :], time_freq[None, :],
      time_phase[None, :])

    mesh = plsc.VectorSubcoreMesh(core_axis_name="c", subcore_axis_name="s")
    sc_fn = pl.kernel(
        _sc_body,
        out_type=(
            jax.ShapeDtypeStruct((NSC, NPAD, DIM), f32),
            jax.ShapeDtypeStruct((NSC, NPAD, DE), f32),
            jax.ShapeDtypeStruct((NSC, NPAD, DE), f32),
        ),
        mesh=mesh,
        scratch_types=[
            pltpu.VMEM((C,), jnp.int32),
            pltpu.VMEM((C,), jnp.int32),
            pltpu.VMEM((CT,), jnp.int32),
            pltpu.VMEM((CT,), jnp.int32),
            pltpu.VMEM((C, DIM), f32),
            pltpu.VMEM((CT, DIM), f32),
            pltpu.VMEM((C, DE), f32),
            pltpu.VMEM((CT, DE), f32),
            pltpu.VMEM((C, DE), f32),
            pltpu.VMEM_SHARED((NPAD, DIM), f32),
            pltpu.VMEM_SHARED((NPAD, DE), f32),
            pltpu.VMEM_SHARED((NPAD, DE), f32),
            pltpu.SemaphoreType.DMA,
        ],
    )
    acc, sat, dega = sc_fn(h_self, src, dst, edge_attr)

    rst_base = pl.pallas_call(
        _rst_base_body,
        grid=(grid1,),
        in_specs=[
            pl.BlockSpec((blk, DIM), lambda i: (i, 0)),
            _full((DIM, DIM)), _full((1, DIM)), _full((1, DIM)),
        ],
        out_specs=pl.BlockSpec((blk, DIM), lambda i: (i, 0)),
        out_shape=jax.ShapeDtypeStruct((NPAD, DIM), f32),
    )(h_self, W_self, b_self[None, :], b_neigh[None, :])

    cblk = 256
    grid4 = NPAD // cblk
    rst = pl.pallas_call(
        _combine_body,
        grid=(grid4,),
        in_specs=[
            pl.BlockSpec((NSC, cblk, DIM), lambda i: (0, i, 0)),
            pl.BlockSpec((NSC, cblk, DE), lambda i: (0, i, 0)),
            pl.BlockSpec((NSC, cblk, DE), lambda i: (0, i, 0)),
            _full((DE, DIM)), _full((1, DIM)), _full((DIM, DIM)),
            pl.BlockSpec((cblk, DIM), lambda i: (i, 0)),
        ],
        out_specs=pl.BlockSpec((cblk, DIM), lambda i: (i, 0)),
        out_shape=jax.ShapeDtypeStruct((NPAD, DIM), f32),
        scratch_shapes=[pltpu.VMEM((1, DIM), f32)],
    )(acc, sat, dega, W_edge, b_edge[None, :], W_neigh, rst_base)

    return rst[:N_NODES]


# speedup vs baseline: 3.1317x; 3.1317x over previous
import jax
import jax.numpy as jnp
from jax import lax
from jax.experimental import pallas as pl
from jax.experimental.pallas import tpu as pltpu
from jax.experimental.pallas import tpu_sc as plsc

N_NODES = 10000
NPAD = 10240
DIM = 128
DE = 16
E_TOTAL = 320000
NSC = 2
NSUB = 16
HDIM = DIM // NSC
EPW = E_TOTAL // NSUB
C = 128
CT = 32
ROWS_PER_SUB = NPAD // NSUB

_HI = lax.Precision.HIGHEST


def _prep_body(x_ref, ts_ref, w1_ref, w2_ref, bt_ref, fr_ref, ph_ref,
               h_ref, lo_ref, hi_ref):
    t_enc = jnp.cos(ts_ref[...] * fr_ref[...] + ph_ref[...])
    h = jnp.dot(x_ref[...], w1_ref[...], preferred_element_type=jnp.float32,
                precision=_HI)
    h += jnp.dot(t_enc, w2_ref[...], preferred_element_type=jnp.float32,
                 precision=_HI)
    h = h + bt_ref[...]
    h_ref[...] = h
    lo_ref[...] = h[:, :HDIM]
    hi_ref[...] = h[:, HDIM:]


def _rst_base_body(h_ref, ws_ref, bs_ref, bn_ref, o_ref):
    o_ref[...] = (jnp.dot(h_ref[...], ws_ref[...],
                          preferred_element_type=jnp.float32, precision=_HI)
                  + bs_ref[...] + bn_ref[...])


def _combine_body(acc_ref, aux_ref, we_ref, be_ref, wn_ref, rb_ref,
                  o_ref, carry_ref):
    i = pl.program_id(0)

    @pl.when(i == 0)
    def _():
        carry_ref[...] = jnp.zeros((1, DIM), jnp.float32)

    a = jnp.concatenate([acc_ref[0], acc_ref[1]], axis=1)
    s = aux_ref[0]
    deg = aux_ref[1, :, 0:1]
    h_ns = a + jnp.dot(s, we_ref[...], preferred_element_type=jnp.float32,
                       precision=_HI) + deg * be_ref[...]
    b = h_ns.shape[0]
    r = lax.broadcasted_iota(jnp.int32, (b, b), 0)
    c = lax.broadcasted_iota(jnp.int32, (b, b), 1)
    tril = (r >= c).astype(jnp.float32)
    cs = jnp.dot(tril, h_ns, preferred_element_type=jnp.float32,
                 precision=_HI) + carry_ref[...]
    carry_ref[...] = cs[b - 1:b, :]
    h_neigh = cs / jnp.maximum(deg, 1.0)
    o_ref[...] = rb_ref[...] + jnp.dot(h_neigh, wn_ref[...],
                                       preferred_element_type=jnp.float32,
                                       precision=_HI)


def _sc_body(hlo_hbm, hhi_hbm, src_hbm, dst_hbm, attr_hbm,
             acc_out, aux_out,
             src_v, dst_v, src_t, dst_t, rows_v, rows_t, attr_v, attr_t,
             ones_v, acc_s, aux_s, sem):
    core = lax.axis_index("c")
    sub = lax.axis_index("s")
    z16 = jnp.zeros((16,), jnp.float32)
    o16 = jnp.ones((16,), jnp.float32)

    @pl.loop(0, C)
    def _(r):
        @pl.loop(0, HDIM, step=16)
        def _(j):
            rows_v[r, pl.ds(j, 16)] = z16

    @pl.loop(0, C)
    def _(r):
        attr_v[r, pl.ds(0, 16)] = z16
        ones_v[r, pl.ds(0, 16)] = o16

    rbase = sub * ROWS_PER_SUB

    @pl.loop(0, ROWS_PER_SUB, step=C)
    def _(k):
        pltpu.sync_copy(rows_v, acc_s.at[pl.ds(rbase + k, C)])
        pltpu.sync_copy(attr_v, aux_s.at[pl.ds(rbase + k, C)])

    plsc.subcore_barrier()

    ebase = sub * EPW

    def chunk(b, n, s_v, d_v, r_v, a_v):
        pltpu.sync_copy(src_hbm.at[pl.ds(b, n)], s_v)
        pltpu.sync_copy(dst_hbm.at[pl.ds(b, n)], d_v)

        @pl.when(core == 0)
        def _():
            pltpu.async_copy(hlo_hbm.at[s_v], r_v, sem).wait()
            pltpu.sync_copy(r_v, acc_s.at[d_v], add=True)
            pltpu.sync_copy(attr_hbm.at[pl.ds(b, n)], a_v)
            pltpu.sync_copy(a_v, aux_s.at[d_v], add=True)

        @pl.when(core == 1)
        def _():
            pltpu.async_copy(hhi_hbm.at[s_v], r_v, sem).wait()
            pltpu.sync_copy(r_v, acc_s.at[d_v], add=True)
            pltpu.sync_copy(ones_v.at[pl.ds(0, n)], aux_s.at[d_v], add=True)

    @pl.loop(0, EPW - CT, step=C)
    def _(i):
        chunk(ebase + i, C, src_v, dst_v, rows_v, attr_v)

    chunk(ebase + (EPW - CT), CT, src_t, dst_t, rows_t, attr_t)

    plsc.subcore_barrier()

    pltpu.sync_copy(acc_s.at[pl.ds(rbase, ROWS_PER_SUB)],
                    acc_out.at[core, pl.ds(rbase, ROWS_PER_SUB)])
    pltpu.sync_copy(aux_s.at[pl.ds(rbase, ROWS_PER_SUB)],
                    aux_out.at[core, pl.ds(rbase, ROWS_PER_SUB)])


def _full(u):
    return pl.BlockSpec(u, lambda *_: tuple(0 for _ in u))


def kernel(x, timestamps, edge_index, edge_attr, new_node_ids,
           time_freq, time_phase, W_time, b_time,
           W_edge, b_edge, W_self, b_self, W_neigh, b_neigh):
    del new_node_ids

    f32 = jnp.float32
    npad = NPAD - N_NODES
    x_p = jnp.pad(x, ((0, npad), (0, 0)))
    ts_p = jnp.pad(jnp.broadcast_to(timestamps[:, None], (N_NODES, DIM)),
                   ((0, npad), (0, 0)))
    src = edge_index[0]
    dst = edge_index[1]
    w1 = W_time[:DIM]
    w2 = W_time[DIM:]

    blk = 1024
    grid1 = NPAD // blk
    h_self, h_lo, h_hi = pl.pallas_call(
        _prep_body,
        grid=(grid1,),
        in_specs=[
            pl.BlockSpec((blk, DIM), lambda i: (i, 0)),
            pl.BlockSpec((blk, DIM), lambda i: (i, 0)),
            _full((DIM, DIM)), _full((DIM, DIM)),
            _full((1, DIM)), _full((1, DIM)), _full((1, DIM)),
        ],
        out_specs=[
            pl.BlockSpec((blk, DIM), lambda i: (i, 0)),
            pl.BlockSpec((blk, HDIM), lambda i: (i, 0)),
            pl.BlockSpec((blk, HDIM), lambda i: (i, 0)),
        ],
        out_shape=[
            jax.ShapeDtypeStruct((NPAD, DIM), f32),
            jax.ShapeDtypeStruct((NPAD, HDIM), f32),
            jax.ShapeDtypeStruct((NPAD, HDIM), f32),
        ],
    )(x_p, ts_p, w1, w2, b_time[None, :], time_freq[None, :],
      time_phase[None, :])

    mesh = plsc.VectorSubcoreMesh(core_axis_name="c", subcore_axis_name="s")
    sc_fn = pl.kernel(
        _sc_body,
        out_type=(
            jax.ShapeDtypeStruct((NSC, NPAD, HDIM), f32),
            jax.ShapeDtypeStruct((NSC, NPAD, DE), f32),
        ),
        mesh=mesh,
        compiler_params=pltpu.CompilerParams(use_tc_tiling_on_sc=False),
        scratch_types=[
            pltpu.VMEM((C,), jnp.int32),
            pltpu.VMEM((C,), jnp.int32),
            pltpu.VMEM((CT,), jnp.int32),
            pltpu.VMEM((CT,), jnp.int32),
            pltpu.VMEM((C, HDIM), f32),
            pltpu.VMEM((CT, HDIM), f32),
            pltpu.VMEM((C, DE), f32),
            pltpu.VMEM((CT, DE), f32),
            pltpu.VMEM((C, DE), f32),
            pltpu.VMEM_SHARED((NPAD, HDIM), f32),
            pltpu.VMEM_SHARED((NPAD, DE), f32),
            pltpu.SemaphoreType.DMA,
        ],
    )
    acc, aux = sc_fn(h_lo, h_hi, src, dst, edge_attr)

    rst_base = pl.pallas_call(
        _rst_base_body,
        grid=(grid1,),
        in_specs=[
            pl.BlockSpec((blk, DIM), lambda i: (i, 0)),
            _full((DIM, DIM)), _full((1, DIM)), _full((1, DIM)),
        ],
        out_specs=pl.BlockSpec((blk, DIM), lambda i: (i, 0)),
        out_shape=jax.ShapeDtypeStruct((NPAD, DIM), f32),
    )(h_self, W_self, b_self[None, :], b_neigh[None, :])

    cblk = 256
    grid4 = NPAD // cblk
    rst = pl.pallas_call(
        _combine_body,
        grid=(grid4,),
        in_specs=[
            pl.BlockSpec((NSC, cblk, HDIM), lambda i: (0, i, 0)),
            pl.BlockSpec((NSC, cblk, DE), lambda i: (0, i, 0)),
            _full((DE, DIM)), _full((1, DIM)), _full((DIM, DIM)),
            pl.BlockSpec((cblk, DIM), lambda i: (i, 0)),
        ],
        out_specs=pl.BlockSpec((cblk, DIM), lambda i: (i, 0)),
        out_shape=jax.ShapeDtypeStruct((NPAD, DIM), f32),
        scratch_shapes=[pltpu.VMEM((1, DIM), f32)],
    )(acc, aux, W_edge, b_edge[None, :], W_neigh, rst_base)

    return rst[:N_NODES]


# speedup vs baseline: 4.9051x; 1.5663x over previous
import jax
import jax.numpy as jnp
from jax import lax
from jax.experimental import pallas as pl
from jax.experimental.pallas import tpu as pltpu
from jax.experimental.pallas import tpu_sc as plsc

N_NODES = 10000
NPAD = 10240
DIM = 128
DE = 16
E_TOTAL = 320000
NSC = 2
NSUB = 16
HDIM = DIM // NSC
EPW = E_TOTAL // NSUB
C = 128
CT = 32
ROWS_PER_SUB = NPAD // NSUB

_HI = lax.Precision.HIGHEST


def _prep_body(x_ref, ts_ref, w1_ref, w2_ref, bt_ref, fr_ref, ph_ref,
               h_ref, lo_ref, hi_ref):
    t_enc = jnp.cos(ts_ref[...] * fr_ref[...] + ph_ref[...])
    h = jnp.dot(x_ref[...], w1_ref[...], preferred_element_type=jnp.float32,
                precision=_HI)
    h += jnp.dot(t_enc, w2_ref[...], preferred_element_type=jnp.float32,
                 precision=_HI)
    h = h + bt_ref[...]
    h_ref[...] = h
    lo_ref[...] = h[:, :HDIM]
    hi_ref[...] = h[:, HDIM:]


def _rst_base_body(h_ref, ws_ref, bs_ref, bn_ref, o_ref):
    o_ref[...] = (jnp.dot(h_ref[...], ws_ref[...],
                          preferred_element_type=jnp.float32, precision=_HI)
                  + bs_ref[...] + bn_ref[...])


def _combine_body(acc_ref, aux_ref, we_ref, be_ref, wn_ref, rb_ref,
                  o_ref, carry_ref):
    i = pl.program_id(0)

    @pl.when(i == 0)
    def _():
        carry_ref[...] = jnp.zeros((1, DIM), jnp.float32)

    a = jnp.concatenate([acc_ref[0], acc_ref[1]], axis=1)
    s = aux_ref[0]
    deg = aux_ref[1, :, 0:1]
    h_ns = a + jnp.dot(s, we_ref[...], preferred_element_type=jnp.float32,
                       precision=_HI) + deg * be_ref[...]
    b = h_ns.shape[0]
    r = lax.broadcasted_iota(jnp.int32, (b, b), 0)
    c = lax.broadcasted_iota(jnp.int32, (b, b), 1)
    tril = (r >= c).astype(jnp.float32)
    cs = jnp.dot(tril, h_ns, preferred_element_type=jnp.float32,
                 precision=_HI) + carry_ref[...]
    carry_ref[...] = cs[b - 1:b, :]
    h_neigh = cs / jnp.maximum(deg, 1.0)
    o_ref[...] = rb_ref[...] + jnp.dot(h_neigh, wn_ref[...],
                                       preferred_element_type=jnp.float32,
                                       precision=_HI)


def _sc_body(hlo_hbm, hhi_hbm, src_hbm, dst_hbm, attr_hbm,
             acc_out, aux_out,
             src_v, dst_v, src_t, dst_t, rows_v, rows_t, attr_v, attr_t,
             ones_v, acc_s, aux_s, semi0, semi1, semg0, semg1, sems0, sems1,
             semt):
    core = lax.axis_index("c")
    sub = lax.axis_index("s")
    z16 = jnp.zeros((16,), jnp.float32)
    o16 = jnp.ones((16,), jnp.float32)


    @pl.loop(0, C)
    def _(r):
        @pl.loop(0, HDIM, step=16)
        def _(j):
            rows_v[0, r, pl.ds(j, 16)] = z16

    @pl.loop(0, C)
    def _(r):
        attr_v[0, r, pl.ds(0, 16)] = z16
        ones_v[r, pl.ds(0, 16)] = o16

    rbase = sub * ROWS_PER_SUB

    @pl.loop(0, ROWS_PER_SUB, step=C)
    def _(k):
        pltpu.sync_copy(rows_v.at[0], acc_s.at[pl.ds(rbase + k, C)])
        pltpu.sync_copy(attr_v.at[0], aux_s.at[pl.ds(rbase + k, C)])

    plsc.subcore_barrier()

    ebase = sub * EPW
    nmain = EPW - CT

    semi = (semi0, semi1)
    semg = (semg0, semg1)
    sems = (sems0, sems1)

    def idx_dmas(i, b):
        yield pltpu.make_async_copy(src_hbm.at[pl.ds(ebase + i, C)],
                                    src_v.at[b], semi[b])
        yield pltpu.make_async_copy(dst_hbm.at[pl.ds(ebase + i, C)],
                                    dst_v.at[b], semi[b])

    def attr_dma(i, b):
        return pltpu.make_async_copy(attr_hbm.at[pl.ds(ebase + i, C)],
                                     attr_v.at[b], semi[b])

    def idx_issue(i, b):
        for d in idx_dmas(i, b):
            d.start()

        @pl.when(core == 0)
        def _():
            attr_dma(i, b).start()

    def idx_wait(i, b):
        for d in idx_dmas(i, b):
            d.wait()

        @pl.when(core == 0)
        def _():
            attr_dma(i, b).wait()

    def gather_dma(b):
        @pl.when(core == 0)
        def _():
            pltpu.make_async_copy(hlo_hbm.at[src_v.at[b]], rows_v.at[b],
                                  semg[b]).start()

        @pl.when(core == 1)
        def _():
            pltpu.make_async_copy(hhi_hbm.at[src_v.at[b]], rows_v.at[b],
                                  semg[b]).start()

    def gather_wait(b):
        pltpu.make_async_copy(hlo_hbm.at[src_v.at[b]], rows_v.at[b],
                              semg[b]).wait()

    def scatter_issue(b):
        pltpu.make_async_copy(rows_v.at[b], acc_s.at[dst_v.at[b]],
                              sems[b]).start(add=True)

        @pl.when(core == 0)
        def _():
            pltpu.make_async_copy(attr_v.at[b], aux_s.at[dst_v.at[b]],
                                  sems[b]).start(add=True)

        @pl.when(core == 1)
        def _():
            pltpu.make_async_copy(ones_v, aux_s.at[dst_v.at[b]],
                                  sems[b]).start(add=True)

    def scatter_wait(b):
        pltpu.make_async_copy(rows_v.at[b], acc_s.at[dst_v.at[b]],
                              sems[b]).wait()
        pltpu.make_async_copy(ones_v, aux_s.at[dst_v.at[b]],
                              sems[b]).wait()

    idx_issue(0, 0)
    idx_issue(C, 1)

    @pl.loop(0, nmain, step=2 * C)
    def _(i):
        idx_wait(i, 0)
        gather_dma(0)
        idx_wait(i + C, 1)
        gather_dma(1)
        gather_wait(0)
        scatter_issue(0)
        gather_wait(1)
        scatter_issue(1)
        scatter_wait(0)

        @pl.when(i + 2 * C < nmain)
        def _():
            idx_issue(i + 2 * C, 0)

        scatter_wait(1)

        @pl.when(i + 3 * C < nmain)
        def _():
            idx_issue(i + 3 * C, 1)

    bt = ebase + nmain
    pltpu.sync_copy(src_hbm.at[pl.ds(bt, CT)], src_t)
    pltpu.sync_copy(dst_hbm.at[pl.ds(bt, CT)], dst_t)

    @pl.when(core == 0)
    def _():
        pltpu.async_copy(hlo_hbm.at[src_t], rows_t, semt).wait()
        pltpu.sync_copy(rows_t, acc_s.at[dst_t], add=True)
        pltpu.sync_copy(attr_hbm.at[pl.ds(bt, CT)], attr_t)
        pltpu.sync_copy(attr_t, aux_s.at[dst_t], add=True)

    @pl.when(core == 1)
    def _():
        pltpu.async_copy(hhi_hbm.at[src_t], rows_t, semt).wait()
        pltpu.sync_copy(rows_t, acc_s.at[dst_t], add=True)
        pltpu.sync_copy(ones_v.at[pl.ds(0, CT)], aux_s.at[dst_t], add=True)

    plsc.subcore_barrier()

    pltpu.sync_copy(acc_s.at[pl.ds(rbase, ROWS_PER_SUB)],
                    acc_out.at[core, pl.ds(rbase, ROWS_PER_SUB)])
    pltpu.sync_copy(aux_s.at[pl.ds(rbase, ROWS_PER_SUB)],
                    aux_out.at[core, pl.ds(rbase, ROWS_PER_SUB)])


def _full(u):
    return pl.BlockSpec(u, lambda *_: tuple(0 for _ in u))


def kernel(x, timestamps, edge_index, edge_attr, new_node_ids,
           time_freq, time_phase, W_time, b_time,
           W_edge, b_edge, W_self, b_self, W_neigh, b_neigh):
    del new_node_ids

    f32 = jnp.float32
    npad = NPAD - N_NODES
    x_p = jnp.pad(x, ((0, npad), (0, 0)))
    ts_p = jnp.pad(jnp.broadcast_to(timestamps[:, None], (N_NODES, DIM)),
                   ((0, npad), (0, 0)))
    src = edge_index[0]
    dst = edge_index[1]
    w1 = W_time[:DIM]
    w2 = W_time[DIM:]

    blk = 1024
    grid1 = NPAD // blk
    h_self, h_lo, h_hi = pl.pallas_call(
        _prep_body,
        grid=(grid1,),
        in_specs=[
            pl.BlockSpec((blk, DIM), lambda i: (i, 0)),
            pl.BlockSpec((blk, DIM), lambda i: (i, 0)),
            _full((DIM, DIM)), _full((DIM, DIM)),
            _full((1, DIM)), _full((1, DIM)), _full((1, DIM)),
        ],
        out_specs=[
            pl.BlockSpec((blk, DIM), lambda i: (i, 0)),
            pl.BlockSpec((blk, HDIM), lambda i: (i, 0)),
            pl.BlockSpec((blk, HDIM), lambda i: (i, 0)),
        ],
        out_shape=[
            jax.ShapeDtypeStruct((NPAD, DIM), f32),
            jax.ShapeDtypeStruct((NPAD, HDIM), f32),
            jax.ShapeDtypeStruct((NPAD, HDIM), f32),
        ],
    )(x_p, ts_p, w1, w2, b_time[None, :], time_freq[None, :],
      time_phase[None, :])

    mesh = plsc.VectorSubcoreMesh(core_axis_name="c", subcore_axis_name="s")
    sc_fn = pl.kernel(
        _sc_body,
        out_type=(
            jax.ShapeDtypeStruct((NSC, NPAD, HDIM), f32),
            jax.ShapeDtypeStruct((NSC, NPAD, DE), f32),
        ),
        mesh=mesh,
        compiler_params=pltpu.CompilerParams(use_tc_tiling_on_sc=False),
        scratch_types=[
            pltpu.VMEM((2, C), jnp.int32),
            pltpu.VMEM((2, C), jnp.int32),
            pltpu.VMEM((CT,), jnp.int32),
            pltpu.VMEM((CT,), jnp.int32),
            pltpu.VMEM((2, C, HDIM), f32),
            pltpu.VMEM((CT, HDIM), f32),
            pltpu.VMEM((2, C, DE), f32),
            pltpu.VMEM((CT, DE), f32),
            pltpu.VMEM((C, DE), f32),
            pltpu.VMEM_SHARED((NPAD, HDIM), f32),
            pltpu.VMEM_SHARED((NPAD, DE), f32),
            pltpu.SemaphoreType.DMA,
            pltpu.SemaphoreType.DMA,
            pltpu.SemaphoreType.DMA,
            pltpu.SemaphoreType.DMA,
            pltpu.SemaphoreType.DMA,
            pltpu.SemaphoreType.DMA,
            pltpu.SemaphoreType.DMA,
        ],
    )
    acc, aux = sc_fn(h_lo, h_hi, src, dst, edge_attr)

    rst_base = pl.pallas_call(
        _rst_base_body,
        grid=(grid1,),
        in_specs=[
            pl.BlockSpec((blk, DIM), lambda i: (i, 0)),
            _full((DIM, DIM)), _full((1, DIM)), _full((1, DIM)),
        ],
        out_specs=pl.BlockSpec((blk, DIM), lambda i: (i, 0)),
        out_shape=jax.ShapeDtypeStruct((NPAD, DIM), f32),
    )(h_self, W_self, b_self[None, :], b_neigh[None, :])

    cblk = 256
    grid4 = NPAD // cblk
    rst = pl.pallas_call(
        _combine_body,
        grid=(grid4,),
        in_specs=[
            pl.BlockSpec((NSC, cblk, HDIM), lambda i: (0, i, 0)),
            pl.BlockSpec((NSC, cblk, DE), lambda i: (0, i, 0)),
            _full((DE, DIM)), _full((1, DIM)), _full((DIM, DIM)),
            pl.BlockSpec((cblk, DIM), lambda i: (i, 0)),
        ],
        out_specs=pl.BlockSpec((cblk, DIM), lambda i: (i, 0)),
        out_shape=jax.ShapeDtypeStruct((NPAD, DIM), f32),
        scratch_shapes=[pltpu.VMEM((1, DIM), f32)],
    )(acc, aux, W_edge, b_edge[None, :], W_neigh, rst_base)

    return rst[:N_NODES]
